# baseline (speedup 1.0000x reference)
import functools

import jax
import jax.numpy as jnp
from jax import lax
from jax.experimental import pallas as pl
from jax.experimental.pallas import tpu as pltpu
from jax.experimental.pallas import tpu_sc as plsc

NC = 2
NS = 16
NW = NC * NS
LANES = 16
GB = 80
SCH = 64


def _mm_body(x_ref, w_ref, o_ref):
    o_ref[...] = jnp.dot(x_ref[...], w_ref[...],
                         preferred_element_type=jnp.float32)


def _add_body(a_ref, b_ref, o_ref):
    o_ref[...] = a_ref[...] + b_ref[...]


def _sc_aggregate(h, src4, dst4, adj2, n, d):
    nst = src4.shape[1]
    ew = nst * SCH * GB
    rpt = (n // NS) // 8 * 8
    rem = n - NS * rpt
    zr = 16
    mesh = plsc.VectorSubcoreMesh(core_axis_name="c", subcore_axis_name="s")

    @functools.partial(
        pl.kernel,
        out_type=jax.ShapeDtypeStruct((NC, n, d), jnp.float32),
        mesh=mesh,
        scratch_types=[
            pltpu.VMEM_SHARED((n, d), jnp.float32),
            pltpu.VMEM((SCH, GB), jnp.int32),
            pltpu.VMEM((SCH, GB), jnp.int32),
            pltpu.VMEM((ew,), jnp.float32),
            pltpu.VMEM((GB, d), jnp.float32),
            pltpu.VMEM((GB, d), jnp.float32),
            pltpu.VMEM((max(zr, rem), d), jnp.float32),
            pltpu.SemaphoreType.DMA,
            pltpu.SemaphoreType.DMA,
        ],
    )
    def k(h_hbm, src_hbm, dst_hbm, adj_hbm, out_hbm,
          acc, srcv, dstv, adjv, rows0, rows1, zbuf, g0, g1):
        c = lax.axis_index("c")
        s = lax.axis_index("s")
        wid = s * NC + c
        bufs = (rows0, rows1)
        gsems = (g0, g1)

        def zrow(i, _):
            for j in range(d // LANES):
                zbuf[i, pl.ds(j * LANES, LANES)] = jnp.zeros(
                    (LANES,), jnp.float32)
            return 0
        lax.fori_loop(0, max(zr, rem), zrow, 0)
        my_base = pl.multiple_of(s * rpt, 8)

        def zcopy(r, _):
            off = pl.multiple_of(s * rpt + r * zr, 8)
            pltpu.sync_copy(zbuf, acc.at[pl.ds(off, zr)])
            return 0
        lax.fori_loop(0, rpt // zr, zcopy, 0)
        if rem:
            @pl.when(s == NS - 1)
            def _():
                pltpu.sync_copy(zbuf.at[pl.ds(0, rem)],
                                acc.at[pl.ds(NS * rpt, rem)])
        plsc.subcore_barrier()

        pltpu.sync_copy(adj_hbm.at[wid], adjv)

        dn = lax.GatherDimensionNumbers(
            offset_dims=(), collapsed_slice_dims=(0,), start_index_map=(0,))

        def scale(buf, g):
            def quad(r, _):
                le = g * GB + r * 4
                acol = pl.multiple_of(le - le % LANES, 8)
                av = adjv[pl.ds(acol, LANES)]
                lane0 = le % LANES
                for t in range(4):
                    sc = lax.gather(
                        av, jnp.full((LANES, 1), lane0 + t, jnp.int32),
                        dn, (1,),
                        mode=lax.GatherScatterMode.PROMISE_IN_BOUNDS)
                    e = r * 4 + t
                    for j in range(d // LANES):
                        sl = pl.ds(j * LANES, LANES)
                        buf[e, sl] = buf[e, sl] * sc
                return 0
            lax.fori_loop(0, GB // 4, quad, 0)

        def window(st, _):
            pltpu.sync_copy(src_hbm.at[wid, st], srcv)
            pltpu.sync_copy(dst_hbm.at[wid, st], dstv)
            def chunk(u, _):
                pltpu.async_copy(h_hbm.at[srcv.at[u]], bufs[0],
                                 gsems[0]).wait()
                scale(bufs[0], st * SCH + u)
                pltpu.sync_copy(bufs[0], acc.at[dstv.at[u]], add=True)
                return 0
            lax.fori_loop(0, SCH, chunk, 0)
            return 0
        lax.fori_loop(0, nst, window, 0)

        plsc.subcore_barrier()
        pltpu.sync_copy(acc.at[pl.ds(my_base, rpt)],
                        out_hbm.at[c, pl.ds(my_base, rpt)])
        if rem:
            @pl.when(s == NS - 1)
            def _():
                pltpu.sync_copy(acc.at[pl.ds(NS * rpt, rem)],
                                out_hbm.at[c, pl.ds(NS * rpt, rem)])

    return k(h, src4, dst4, adj2)


def kernel(x, edge_index, adj_values, w):
    n, d_in = x.shape
    d_out = w.shape[1]
    e = adj_values.shape[0]

    bm = 1000
    nb = n // bm
    h = pl.pallas_call(
        _mm_body,
        grid=(nb,),
        in_specs=[
            pl.BlockSpec((bm, d_in), lambda i: (i, 0)),
            pl.BlockSpec((d_in, d_out), lambda i: (0, 0)),
        ],
        out_specs=pl.BlockSpec((bm, d_out), lambda i: (i, 0)),
        out_shape=jax.ShapeDtypeStruct((n, d_out), jnp.float32),
    )(x, w)

    dst = edge_index[0]
    src = edge_index[1]
    span = NW * GB * SCH
    e_pad = (e + span - 1) // span * span
    if e_pad != e:
        pad = e_pad - e
        src = jnp.concatenate([src, jnp.zeros((pad,), jnp.int32)])
        dst = jnp.concatenate([dst, jnp.zeros((pad,), jnp.int32)])
        adj_values = jnp.concatenate(
            [adj_values, jnp.zeros((pad,), jnp.float32)])
    ew = e_pad // NW
    nst = ew // (SCH * GB)
    src4 = src.reshape(NW, nst, SCH, GB)
    dst4 = dst.reshape(NW, nst, SCH, GB)
    adj2 = adj_values.reshape(NW, ew)

    partial = _sc_aggregate(h, src4, dst4, adj2, n, d_out)

    out = pl.pallas_call(
        _add_body,
        grid=(nb,),
        in_specs=[
            pl.BlockSpec((bm, d_out), lambda i: (i, 0)),
            pl.BlockSpec((bm, d_out), lambda i: (i, 0)),
        ],
        out_specs=pl.BlockSpec((bm, d_out), lambda i: (i, 0)),
        out_shape=jax.ShapeDtypeStruct((n, d_out), jnp.float32),
    )(partial[0], partial[1])
    return out


# speedup vs baseline: 2.3979x; 2.3979x over previous
import functools

import jax
import jax.numpy as jnp
from jax import lax
from jax.experimental import pallas as pl
from jax.experimental.pallas import tpu as pltpu
from jax.experimental.pallas import tpu_sc as plsc

NC = 2
NS = 16
NW = NC * NS
LANES = 16
GB = 80


def _mm_body(x_ref, w_ref, o_ref):
    o_ref[...] = jnp.dot(x_ref[...], w_ref[...],
                         preferred_element_type=jnp.float32)


def _add_body(a_ref, b_ref, o_ref):
    o_ref[...] = a_ref[...] + b_ref[...]


def _sc_aggregate(h, src2, dst2, adj2, n, d):
    nst, sch = src2.shape[1], src2.shape[2]
    ew = nst * sch * GB
    rpt = (n // NS) // 8 * 8
    rem = n - NS * rpt
    zr = 16
    mesh = plsc.VectorSubcoreMesh(core_axis_name="c", subcore_axis_name="s")

    @functools.partial(
        pl.kernel,
        out_type=jax.ShapeDtypeStruct((NC, n, d), jnp.float32),
        mesh=mesh,
        scratch_types=[
            pltpu.VMEM_SHARED((n, d), jnp.float32),
            pltpu.VMEM((sch, GB), jnp.int32),
            pltpu.VMEM((sch, GB), jnp.int32),
            pltpu.VMEM((ew,), jnp.float32),
            pltpu.VMEM((GB, d), jnp.float32),
            pltpu.VMEM((GB, d), jnp.float32),
            pltpu.VMEM((max(zr, rem), d), jnp.float32),
            pltpu.SemaphoreType.DMA,
            pltpu.SemaphoreType.DMA,
        ],
    )
    def k(h_hbm, src_hbm, dst_hbm, adj_hbm, out_hbm,
          acc, srcv, dstv, adjv, rows0, rows1, zbuf, g0, g1):
        c = lax.axis_index("c")
        s = lax.axis_index("s")
        wid = s * NC + c
        bufs = (rows0, rows1)
        gsems = (g0, g1)

        def zrow(i, _):
            for j in range(d // LANES):
                zbuf[i, pl.ds(j * LANES, LANES)] = jnp.zeros(
                    (LANES,), jnp.float32)
            return 0
        lax.fori_loop(0, max(zr, rem), zrow, 0)
        my_base = pl.multiple_of(s * rpt, 8)

        def zcopy(r, _):
            off = pl.multiple_of(s * rpt + r * zr, 8)
            pltpu.sync_copy(zbuf, acc.at[pl.ds(off, zr)])
            return 0
        lax.fori_loop(0, rpt // zr, zcopy, 0)
        if rem:
            @pl.when(s == NS - 1)
            def _():
                pltpu.sync_copy(zbuf.at[pl.ds(0, rem)],
                                acc.at[pl.ds(NS * rpt, rem)])
        plsc.subcore_barrier()

        pltpu.sync_copy(adj_hbm.at[wid], adjv)

        dn = lax.GatherDimensionNumbers(
            offset_dims=(), collapsed_slice_dims=(0,), start_index_map=(0,))

        def scale(buf, g):
            def grp(q, _):
                av = adjv[pl.ds(g * GB + q * LANES, LANES)]

                def row(t, _):
                    e = q * LANES + t
                    sc = lax.gather(
                        av, jnp.full((LANES, 1), t, jnp.int32), dn, (1,),
                        mode=lax.GatherScatterMode.PROMISE_IN_BOUNDS)
                    for j in range(d // LANES):
                        sl = pl.ds(j * LANES, LANES)
                        buf[e, sl] = buf[e, sl] * sc
                    return 0
                lax.fori_loop(0, LANES, row, 0)
                return 0
            lax.fori_loop(0, GB // LANES, grp, 0)

        def stage(st, _):
            pltpu.sync_copy(src_hbm.at[wid, st], srcv)
            pltpu.sync_copy(dst_hbm.at[wid, st], dstv)
            pltpu.async_copy(h_hbm.at[srcv.at[0]], bufs[0], gsems[0]).wait()

            def pair(p, _):
                for b in range(2):
                    u = p * 2 + b
                    nb = 1 - b
                    cp = pltpu.async_copy(h_hbm.at[srcv.at[u + 1]],
                                          bufs[nb], gsems[nb])
                    scale(bufs[b], st * sch + u)
                    cp.wait()
                    pltpu.sync_copy(bufs[b], acc.at[dstv.at[u]], add=True)
                return 0
            lax.fori_loop(0, (sch - 1) // 2, pair, 0)
            scale(bufs[0], st * sch + sch - 1)
            pltpu.sync_copy(bufs[0], acc.at[dstv.at[sch - 1]], add=True)
            return 0
        lax.fori_loop(0, nst, stage, 0)

        plsc.subcore_barrier()
        pltpu.sync_copy(acc.at[pl.ds(my_base, rpt)],
                        out_hbm.at[c, pl.ds(my_base, rpt)])
        if rem:
            @pl.when(s == NS - 1)
            def _():
                pltpu.sync_copy(acc.at[pl.ds(NS * rpt, rem)],
                                out_hbm.at[c, pl.ds(NS * rpt, rem)])

    return k(h, src2, dst2, adj2)


def kernel(x, edge_index, adj_values, w):
    n, d_in = x.shape
    d_out = w.shape[1]
    e = adj_values.shape[0]

    bm = 1000
    nb = n // bm
    h = pl.pallas_call(
        _mm_body,
        grid=(nb,),
        in_specs=[
            pl.BlockSpec((bm, d_in), lambda i: (i, 0)),
            pl.BlockSpec((d_in, d_out), lambda i: (0, 0)),
        ],
        out_specs=pl.BlockSpec((bm, d_out), lambda i: (i, 0)),
        out_shape=jax.ShapeDtypeStruct((n, d_out), jnp.float32),
    )(x, w)

    dst = edge_index[0]
    src = edge_index[1]
    span = NW * GB
    e_pad = (e + span - 1) // span * span
    if e_pad != e:
        pad = e_pad - e
        src = jnp.concatenate([src, jnp.zeros((pad,), jnp.int32)])
        dst = jnp.concatenate([dst, jnp.zeros((pad,), jnp.int32)])
        adj_values = jnp.concatenate(
            [adj_values, jnp.zeros((pad,), jnp.float32)])
    ew = e_pad // NW
    ng = ew // GB
    sch = next(c for c in (25, 20, 16, 10, 8, 5, 4, 2, 1) if ng % c == 0)
    src2 = src.reshape(NW, ng // sch, sch, GB)
    dst2 = dst.reshape(NW, ng // sch, sch, GB)
    adj2 = adj_values.reshape(NW, ew)

    partial = _sc_aggregate(h, src2, dst2, adj2, n, d_out)

    out = pl.pallas_call(
        _add_body,
        grid=(nb,),
        in_specs=[
            pl.BlockSpec((bm, d_out), lambda i: (i, 0)),
            pl.BlockSpec((bm, d_out), lambda i: (i, 0)),
        ],
        out_specs=pl.BlockSpec((bm, d_out), lambda i: (i, 0)),
        out_shape=jax.ShapeDtypeStruct((n, d_out), jnp.float32),
    )(partial[0], partial[1])
    return out


# speedup vs baseline: 2.4389x; 1.0171x over previous
import functools

import jax
import jax.numpy as jnp
from jax import lax
from jax.experimental import pallas as pl
from jax.experimental.pallas import tpu as pltpu
from jax.experimental.pallas import tpu_sc as plsc

NC = 2
NS = 16
NW = NC * NS
LANES = 16
GB = 80


def _mm_body(x_ref, w_ref, o_ref):
    o_ref[...] = jnp.dot(x_ref[...], w_ref[...],
                         preferred_element_type=jnp.float32)


def _add_body(a_ref, b_ref, o_ref):
    o_ref[...] = a_ref[...] + b_ref[...]


def _sc_aggregate(h, src2, dst2, adj2, n, d):
    nst, sch = src2.shape[1], src2.shape[2]
    ew = nst * sch * GB
    rpt = (n // NS) // 8 * 8
    rem = n - NS * rpt
    zr = 16
    mesh = plsc.VectorSubcoreMesh(core_axis_name="c", subcore_axis_name="s")

    @functools.partial(
        pl.kernel,
        out_type=jax.ShapeDtypeStruct((NC, n, d), jnp.float32),
        mesh=mesh,
        scratch_types=[
            pltpu.VMEM_SHARED((n, d), jnp.float32),
            pltpu.VMEM((sch, GB), jnp.int32),
            pltpu.VMEM((sch, GB), jnp.int32),
            pltpu.VMEM((ew,), jnp.float32),
            pltpu.VMEM((GB, d), jnp.float32),
            pltpu.VMEM((GB, d), jnp.float32),
            pltpu.VMEM((max(zr, rem), d), jnp.float32),
            pltpu.SemaphoreType.DMA,
            pltpu.SemaphoreType.DMA,
        ],
    )
    def k(h_hbm, src_hbm, dst_hbm, adj_hbm, out_hbm,
          acc, srcv, dstv, adjv, rows0, rows1, zbuf, g0, g1):
        c = lax.axis_index("c")
        s = lax.axis_index("s")
        wid = s * NC + c
        bufs = (rows0, rows1)
        gsems = (g0, g1)

        def zrow(i, _):
            for j in range(d // LANES):
                zbuf[i, pl.ds(j * LANES, LANES)] = jnp.zeros(
                    (LANES,), jnp.float32)
            return 0
        lax.fori_loop(0, max(zr, rem), zrow, 0)
        my_base = pl.multiple_of(s * rpt, 8)

        def zcopy(r, _):
            off = pl.multiple_of(s * rpt + r * zr, 8)
            pltpu.sync_copy(zbuf, acc.at[pl.ds(off, zr)])
            return 0
        lax.fori_loop(0, rpt // zr, zcopy, 0)
        if rem:
            @pl.when(s == NS - 1)
            def _():
                pltpu.sync_copy(zbuf.at[pl.ds(0, rem)],
                                acc.at[pl.ds(NS * rpt, rem)])
        plsc.subcore_barrier()

        pltpu.sync_copy(adj_hbm.at[wid], adjv)

        dn = lax.GatherDimensionNumbers(
            offset_dims=(), collapsed_slice_dims=(0,), start_index_map=(0,))

        def scale(buf, g):
            def grp(q, _):
                av = adjv[pl.ds(g * GB + q * LANES, LANES)]

                def row(t, _):
                    e = q * LANES + t
                    sc = lax.gather(
                        av, jnp.full((LANES, 1), t, jnp.int32), dn, (1,),
                        mode=lax.GatherScatterMode.PROMISE_IN_BOUNDS)
                    for j in range(d // LANES):
                        sl = pl.ds(j * LANES, LANES)
                        buf[e, sl] = buf[e, sl] * sc
                    return 0
                lax.fori_loop(0, LANES, row, 0)
                return 0
            lax.fori_loop(0, GB // LANES, grp, 0)

        def stage(st, _):
            pltpu.sync_copy(src_hbm.at[wid, st], srcv)
            pltpu.sync_copy(dst_hbm.at[wid, st], dstv)
            pltpu.async_copy(h_hbm.at[srcv.at[0]], bufs[0], gsems[0]).wait()

            def pair(p, _):
                for b in range(2):
                    u = p * 2 + b
                    nb = 1 - b
                    cp = pltpu.async_copy(h_hbm.at[srcv.at[u + 1]],
                                          bufs[nb], gsems[nb])
                    scale(bufs[b], st * sch + u)
                    cp.wait()
                    pltpu.sync_copy(bufs[b], acc.at[dstv.at[u]], add=True)
                return 0
            lax.fori_loop(0, (sch - 1) // 2, pair, 0)
            scale(bufs[0], st * sch + sch - 1)
            pltpu.sync_copy(bufs[0], acc.at[dstv.at[sch - 1]], add=True)
            return 0
        lax.fori_loop(0, nst, stage, 0)

        plsc.subcore_barrier()
        pltpu.sync_copy(acc.at[pl.ds(my_base, rpt)],
                        out_hbm.at[c, pl.ds(my_base, rpt)])
        if rem:
            @pl.when(s == NS - 1)
            def _():
                pltpu.sync_copy(acc.at[pl.ds(NS * rpt, rem)],
                                out_hbm.at[c, pl.ds(NS * rpt, rem)])

    return k(h, src2, dst2, adj2)


def kernel(x, edge_index, adj_values, w):
    n, d_in = x.shape
    d_out = w.shape[1]
    e = adj_values.shape[0]

    bm = 2000
    nb = n // bm
    h = pl.pallas_call(
        _mm_body,
        grid=(nb,),
        in_specs=[
            pl.BlockSpec((bm, d_in), lambda i: (i, 0)),
            pl.BlockSpec((d_in, d_out), lambda i: (0, 0)),
        ],
        out_specs=pl.BlockSpec((bm, d_out), lambda i: (i, 0)),
        out_shape=jax.ShapeDtypeStruct((n, d_out), jnp.float32),
    )(x, w)

    dst = edge_index[0]
    src = edge_index[1]
    span = NW * GB
    e_pad = (e + span - 1) // span * span
    if e_pad != e:
        pad = e_pad - e
        src = jnp.concatenate([src, jnp.zeros((pad,), jnp.int32)])
        dst = jnp.concatenate([dst, jnp.zeros((pad,), jnp.int32)])
        adj_values = jnp.concatenate(
            [adj_values, jnp.zeros((pad,), jnp.float32)])
    ew = e_pad // NW
    ng = ew // GB
    sch = next(c for c in (25, 20, 16, 10, 8, 5, 4, 2, 1) if ng % c == 0)
    src2 = src.reshape(NW, ng // sch, sch, GB)
    dst2 = dst.reshape(NW, ng // sch, sch, GB)
    adj2 = adj_values.reshape(NW, ew)

    partial = _sc_aggregate(h, src2, dst2, adj2, n, d_out)

    out = pl.pallas_call(
        _add_body,
        grid=(nb,),
        in_specs=[
            pl.BlockSpec((bm, d_out), lambda i: (i, 0)),
            pl.BlockSpec((bm, d_out), lambda i: (i, 0)),
        ],
        out_specs=pl.BlockSpec((bm, d_out), lambda i: (i, 0)),
        out_shape=jax.ShapeDtypeStruct((n, d_out), jnp.float32),
    )(partial[0], partial[1])
    return out
